# baseline (speedup 1.0000x reference)
import math

import jax
import jax.numpy as jnp
from jax.experimental import pallas as pl
from jax.experimental.pallas import tpu as pltpu

_N = 4096
_T = 0.1
_BLK = 256
_G = _N // _BLK
_NOFF = _G * (_G - 1)
_STEPS = _G * _G
_C = _N // _G

_BETA = 2.0 / (_N * _T)
_ZS = sum(math.exp(-_BETA * min(k + 1, _N - 1 - k)) for k in range(_N - 1))
_INV_ZS = 1.0 / _ZS
_NF = float(_N)


def _w_of_k(k):
    kk = jnp.minimum(k + 1.0, _NF - 1.0 - k)
    val = jnp.exp(-_BETA * kk) * _INV_ZS
    return jnp.where((k >= 0.0) & (k <= _NF - 2.0), val, 0.0)


def _fused_kernel(blk_ref, out_ref, row_sc, col_sc, md_sc, acc_ref):
    s = pl.program_id(0)

    @pl.when(s < _G)
    def _gather():
        blk = blk_ref[...]
        r_io = jax.lax.broadcasted_iota(jnp.int32, (_BLK, _BLK), 0)
        c_io = jax.lax.broadcasted_iota(jnp.int32, (_BLK, _BLK), 1)
        dblk = jnp.where(r_io == c_io, blk, 0.0)
        row_sc[:, pl.ds(s * _BLK, _BLK)] = jnp.sum(dblk, axis=0, keepdims=True)
        col_sc[pl.ds(s * _BLK, _BLK), :] = jnp.sum(dblk, axis=1, keepdims=True)

    @pl.when((s >= _G) & (s < 2 * _G))
    def _rank_chunk():
        ci = s - _G
        d_row = row_sc[...]
        d_col = col_sc[pl.ds(ci * _C, _C), :]
        lt = (d_row < d_col).astype(jnp.float32)
        ones = jnp.ones((_N, 1), dtype=jnp.float32)
        rank = jax.lax.dot_general(
            lt, ones, (((1,), (0,)), ((), ())),
            preferred_element_type=jnp.float32)
        cval = 0.5 * (_w_of_k(rank - 1.0) + _w_of_k(rank))
        partial = jnp.sum(d_col * cval)
        acc_ref[0, 0] = jnp.where(ci == 0, partial, acc_ref[0, 0] + partial)

    @pl.when(s == 2 * _G)
    def _finalize():
        thr = acc_ref[0, 0]
        d_row = row_sc[...]
        md_sc[...] = jnp.where(d_row > thr, 0.0, d_row)

    @pl.when(s < _NOFF)
    def _zeros():
        out_ref[...] = jnp.zeros((_BLK, _BLK), jnp.float32)

    @pl.when(s >= _NOFF)
    def _diag_block():
        i = s - _NOFF
        chunk = md_sc[:, pl.ds(i * _BLK, _BLK)]
        r_io = jax.lax.broadcasted_iota(jnp.int32, (_BLK, _BLK), 0)
        c_io = jax.lax.broadcasted_iota(jnp.int32, (_BLK, _BLK), 1)
        out_ref[...] = jnp.where(r_io == c_io,
                                 jnp.broadcast_to(chunk, (_BLK, _BLK)), 0.0)


def _in_map(s):
    i = jnp.minimum(s, _G - 1)
    return (i, i)


def _out_map(s):
    r = s // (_G - 1)
    m = s % (_G - 1)
    bj_off = m + (m >= r).astype(jnp.int32)
    d = s - _NOFF
    bi = jnp.where(s < _NOFF, r, d)
    bj = jnp.where(s < _NOFF, bj_off, d)
    return (bi, bj)


@jax.jit
def kernel(diagonal_matrix):
    return pl.pallas_call(
        _fused_kernel,
        grid=(_STEPS,),
        in_specs=[pl.BlockSpec((_BLK, _BLK), _in_map)],
        out_specs=pl.BlockSpec((_BLK, _BLK), _out_map),
        out_shape=jax.ShapeDtypeStruct((_N, _N), jnp.float32),
        scratch_shapes=[
            pltpu.VMEM((1, _N), jnp.float32),
            pltpu.VMEM((_N, 1), jnp.float32),
            pltpu.VMEM((1, _N), jnp.float32),
            pltpu.SMEM((1, 1), jnp.float32),
        ],
        compiler_params=pltpu.CompilerParams(
            dimension_semantics=("arbitrary",),
        ),
    )(diagonal_matrix)


# speedup vs baseline: 2.7633x; 2.7633x over previous
import math

import jax
import jax.numpy as jnp
from jax.experimental import pallas as pl
from jax.experimental.pallas import tpu as pltpu

_N = 4096
_T = 0.1
_BD = 256
_GD = _N // _BD
_C = 256
_BR = 1024
_GR = _N // _BR
_RANK_STEP = _GD
_STEPS = _GD + 1 + _GR

_BETA = 2.0 / (_N * _T)
_ZS = sum(math.exp(-_BETA * min(k + 1, _N - 1 - k)) for k in range(_N - 1))
_INV_ZS = 1.0 / _ZS
_NF = float(_N)


def _w_of_k(k):
    kk = jnp.minimum(k + 1.0, _NF - 1.0 - k)
    val = jnp.exp(-_BETA * kk) * _INV_ZS
    return jnp.where((k >= 0.0) & (k <= _NF - 2.0), val, 0.0)


def _fused_kernel(blk_ref, out_ref, row_sc, col_sc, md_sc):
    s = pl.program_id(0)

    @pl.when(s < _GD)
    def _gather():
        blk = blk_ref[...]
        r_io = jax.lax.broadcasted_iota(jnp.int32, (_BD, _BD), 0)
        c_io = jax.lax.broadcasted_iota(jnp.int32, (_BD, _BD), 1)
        dblk = jnp.where(r_io == c_io, blk, 0.0)
        row_sc[:, pl.ds(s * _BD, _BD)] = jnp.sum(dblk, axis=0, keepdims=True)
        col_sc[pl.ds(s * _BD, _BD), :] = jnp.sum(dblk, axis=1, keepdims=True)

    @pl.when(s == _RANK_STEP)
    def _threshold():
        d_row = row_sc[...]
        ones = jnp.ones((_N, 1), dtype=jnp.float32)
        thr = jnp.float32(0.0)
        for ci in range(_N // _C):
            d_col = col_sc[ci * _C:(ci + 1) * _C, :]
            lt = (d_row < d_col).astype(jnp.float32)
            rank = jax.lax.dot_general(
                lt, ones, (((1,), (0,)), ((), ())),
                preferred_element_type=jnp.float32)
            cval = 0.5 * (_w_of_k(rank - 1.0) + _w_of_k(rank))
            thr = thr + jnp.sum(d_col * cval)
        md_sc[...] = jnp.where(d_row > thr, 0.0, d_row)

    @pl.when(s > _RANK_STEP)
    def _write():
        slab = s - _RANK_STEP - 1
        r_io = jax.lax.broadcasted_iota(jnp.int32, (_BR, _N), 0)
        c_io = jax.lax.broadcasted_iota(jnp.int32, (_BR, _N), 1)
        mask = c_io == r_io + slab * _BR
        out_ref[...] = jnp.where(mask, md_sc[...], 0.0)


def _in_map(s):
    i = jnp.minimum(s, _GD - 1)
    return (i, i)


def _out_map(s):
    return (jnp.maximum(s - _RANK_STEP - 1, 0), 0)


@jax.jit
def kernel(diagonal_matrix):
    return pl.pallas_call(
        _fused_kernel,
        grid=(_STEPS,),
        in_specs=[pl.BlockSpec((_BD, _BD), _in_map)],
        out_specs=pl.BlockSpec((_BR, _N), _out_map),
        out_shape=jax.ShapeDtypeStruct((_N, _N), jnp.float32),
        scratch_shapes=[
            pltpu.VMEM((1, _N), jnp.float32),
            pltpu.VMEM((_N, 1), jnp.float32),
            pltpu.VMEM((1, _N), jnp.float32),
        ],
        compiler_params=pltpu.CompilerParams(
            dimension_semantics=("arbitrary",),
        ),
    )(diagonal_matrix)


# speedup vs baseline: 3.2805x; 1.1872x over previous
import math

import jax
import jax.numpy as jnp
from jax.experimental import pallas as pl
from jax.experimental.pallas import tpu as pltpu

_N = 4096
_T = 0.1
_BD = 256
_NIN = 4
_GD = _N // (_BD * _NIN)
_C = 256
_BR = 512
_GR = _N // _BR
_RANK_STEP = _GD
_STEPS = _GD + 1 + _GR

_BETA = 2.0 / (_N * _T)
_ZS = sum(math.exp(-_BETA * min(k + 1, _N - 1 - k)) for k in range(_N - 1))
_INV_ZS = 1.0 / _ZS
_NF = float(_N)


def _w_of_k(k):
    kk = jnp.minimum(k + 1.0, _NF - 1.0 - k)
    val = jnp.exp(-_BETA * kk) * _INV_ZS
    return jnp.where((k >= 0.0) & (k <= _NF - 2.0), val, 0.0)


def _fused_kernel(b0_ref, b1_ref, b2_ref, b3_ref, out_ref, row_sc, col_sc,
                  md_sc):
    s = pl.program_id(0)
    blk_refs = (b0_ref, b1_ref, b2_ref, b3_ref)

    @pl.when(s < _GD)
    def _gather():
        r_io = jax.lax.broadcasted_iota(jnp.int32, (_BD, _BD), 0)
        c_io = jax.lax.broadcasted_iota(jnp.int32, (_BD, _BD), 1)
        eye = r_io == c_io
        for k, bref in enumerate(blk_refs):
            dblk = jnp.where(eye, bref[...], 0.0)
            base = (s * _NIN + k) * _BD
            row_sc[:, pl.ds(base, _BD)] = jnp.sum(dblk, axis=0, keepdims=True)
            col_sc[pl.ds(base, _BD), :] = jnp.sum(dblk, axis=1, keepdims=True)

    @pl.when(s == _RANK_STEP)
    def _threshold():
        d_row = row_sc[...]
        ones = jnp.ones((_N, 1), dtype=jnp.float32)
        thr = jnp.float32(0.0)
        for ci in range(_N // _C):
            d_col = col_sc[ci * _C:(ci + 1) * _C, :]
            lt = (d_row < d_col).astype(jnp.float32)
            rank = jax.lax.dot_general(
                lt, ones, (((1,), (0,)), ((), ())),
                preferred_element_type=jnp.float32)
            cval = 0.5 * (_w_of_k(rank - 1.0) + _w_of_k(rank))
            thr = thr + jnp.sum(d_col * cval)
        md_sc[...] = jnp.where(d_row > thr, 0.0, d_row)

    @pl.when(s > _RANK_STEP)
    def _write():
        slab = s - _RANK_STEP - 1
        r_io = jax.lax.broadcasted_iota(jnp.int32, (_BR, _N), 0)
        c_io = jax.lax.broadcasted_iota(jnp.int32, (_BR, _N), 1)
        mask = c_io == r_io + slab * _BR
        out_ref[...] = jnp.where(mask, md_sc[...], 0.0)


def _make_in_map(k):
    last = (_GD - 1) * _NIN + k

    def in_map(s):
        i = jnp.minimum(s * _NIN + k, last)
        return (i, i)

    return in_map


def _out_map(s):
    return (jnp.maximum(s - _RANK_STEP - 1, 0), 0)


@jax.jit
def kernel(diagonal_matrix):
    return pl.pallas_call(
        _fused_kernel,
        grid=(_STEPS,),
        in_specs=[
            pl.BlockSpec((_BD, _BD), _make_in_map(k)) for k in range(_NIN)
        ],
        out_specs=pl.BlockSpec((_BR, _N), _out_map),
        out_shape=jax.ShapeDtypeStruct((_N, _N), jnp.float32),
        scratch_shapes=[
            pltpu.VMEM((1, _N), jnp.float32),
            pltpu.VMEM((_N, 1), jnp.float32),
            pltpu.VMEM((1, _N), jnp.float32),
        ],
        compiler_params=pltpu.CompilerParams(
            dimension_semantics=("arbitrary",),
        ),
    )(diagonal_matrix, diagonal_matrix, diagonal_matrix, diagonal_matrix)


# speedup vs baseline: 3.2984x; 1.0055x over previous
import math

import jax
import jax.numpy as jnp
from jax.experimental import pallas as pl
from jax.experimental.pallas import tpu as pltpu

_N = 4096
_T = 0.1
_BD = 256
_NIN = 4
_GD = _N // (_BD * _NIN)
_BR = 512
_GR = _N // _BR
_C = 512
_STEPS = _GD + _GR + 1
_DB = 256
_GB = _N // _DB

_BETA = 2.0 / (_N * _T)
_ZS = sum(math.exp(-_BETA * min(k + 1, _N - 1 - k)) for k in range(_N - 1))
_INV_ZS = 1.0 / _ZS
_NF = float(_N)


def _w_of_k(k):
    kk = jnp.minimum(k + 1.0, _NF - 1.0 - k)
    val = jnp.exp(-_BETA * kk) * _INV_ZS
    return jnp.where((k >= 0.0) & (k <= _NF - 2.0), val, 0.0)


def _zeros_rank_kernel(b0_ref, b1_ref, b2_ref, b3_ref, out_ref, md_ref,
                       row_sc, col_sc, acc_ref):
    s = pl.program_id(0)
    blk_refs = (b0_ref, b1_ref, b2_ref, b3_ref)

    @pl.when(s < _GD)
    def _gather():
        r_io = jax.lax.broadcasted_iota(jnp.int32, (_BD, _BD), 0)
        c_io = jax.lax.broadcasted_iota(jnp.int32, (_BD, _BD), 1)
        eye = r_io == c_io
        for k, bref in enumerate(blk_refs):
            dblk = jnp.where(eye, bref[...], 0.0)
            base = (s * _NIN + k) * _BD
            row_sc[:, pl.ds(base, _BD)] = jnp.sum(dblk, axis=0, keepdims=True)
            col_sc[pl.ds(base, _BD), :] = jnp.sum(dblk, axis=1, keepdims=True)

    @pl.when((s >= _GD) & (s < _GD + _GR))
    def _zeros_and_rank():
        out_ref[...] = jnp.zeros((_BR, _N), jnp.float32)
        ci = s - _GD
        d_row = row_sc[...]
        d_col = col_sc[pl.ds(ci * _C, _C), :]
        lt = (d_row < d_col).astype(jnp.float32)
        ones = jnp.ones((_N, 1), dtype=jnp.float32)
        rank = jax.lax.dot_general(
            lt, ones, (((1,), (0,)), ((), ())),
            preferred_element_type=jnp.float32)
        cval = 0.5 * (_w_of_k(rank - 1.0) + _w_of_k(rank))
        partial = jnp.sum(d_col * cval)
        acc_ref[0, 0] = jnp.where(ci == 0, partial, acc_ref[0, 0] + partial)

    @pl.when(s == _STEPS - 1)
    def _finalize():
        thr = acc_ref[0, 0]
        d_row = row_sc[...]
        md_ref[...] = jnp.where(d_row > thr, 0.0, d_row)


def _diag_update_kernel(md_ref, zeros_ref, out_ref):
    del zeros_ref
    i = pl.program_id(0)
    chunk = md_ref[:, pl.ds(i * _DB, _DB)]
    r_io = jax.lax.broadcasted_iota(jnp.int32, (_DB, _DB), 0)
    c_io = jax.lax.broadcasted_iota(jnp.int32, (_DB, _DB), 1)
    out_ref[...] = jnp.where(r_io == c_io,
                             jnp.broadcast_to(chunk, (_DB, _DB)), 0.0)


def _make_in_map(k):
    last = (_GD - 1) * _NIN + k

    def in_map(s):
        i = jnp.minimum(s * _NIN + k, last)
        return (i, i)

    return in_map


def _slab_map(s):
    return (jnp.clip(s - _GD, 0, _GR - 1), 0)


@jax.jit
def kernel(diagonal_matrix):
    zeros_mat, md = pl.pallas_call(
        _zeros_rank_kernel,
        grid=(_STEPS,),
        in_specs=[
            pl.BlockSpec((_BD, _BD), _make_in_map(k)) for k in range(_NIN)
        ],
        out_specs=[
            pl.BlockSpec((_BR, _N), _slab_map),
            pl.BlockSpec((1, _N), lambda s: (0, 0)),
        ],
        out_shape=[
            jax.ShapeDtypeStruct((_N, _N), jnp.float32),
            jax.ShapeDtypeStruct((1, _N), jnp.float32),
        ],
        scratch_shapes=[
            pltpu.VMEM((1, _N), jnp.float32),
            pltpu.VMEM((_N, 1), jnp.float32),
            pltpu.SMEM((1, 1), jnp.float32),
        ],
        compiler_params=pltpu.CompilerParams(
            dimension_semantics=("arbitrary",),
        ),
    )(diagonal_matrix, diagonal_matrix, diagonal_matrix, diagonal_matrix)

    out = pl.pallas_call(
        _diag_update_kernel,
        grid=(_GB,),
        in_specs=[
            pl.BlockSpec((1, _N), lambda i: (0, 0)),
            pl.BlockSpec((8, 128), lambda i: (0, 0)),
        ],
        out_specs=pl.BlockSpec((_DB, _DB), lambda i: (i, i)),
        out_shape=jax.ShapeDtypeStruct((_N, _N), jnp.float32),
        input_output_aliases={1: 0},
        compiler_params=pltpu.CompilerParams(
            dimension_semantics=("arbitrary",),
        ),
    )(md, zeros_mat)
    return out


# speedup vs baseline: 3.3198x; 1.0065x over previous
import math

import jax
import jax.numpy as jnp
from jax.experimental import pallas as pl
from jax.experimental.pallas import tpu as pltpu

_N = 4096
_T = 0.1
_BD = 256
_NIN = 8
_GD = _N // (_BD * _NIN)
_C = 1024
_BR = 512
_GR = _N // _BR
_RANK_STEP = _GD
_STEPS = _GD + 1 + _GR

_BETA = 2.0 / (_N * _T)
_ZS = sum(math.exp(-_BETA * min(k + 1, _N - 1 - k)) for k in range(_N - 1))
_INV_ZS = 1.0 / _ZS
_NF = float(_N)


def _w_of_k(k):
    kk = jnp.minimum(k + 1.0, _NF - 1.0 - k)
    val = jnp.exp(-_BETA * kk) * _INV_ZS
    return jnp.where((k >= 0.0) & (k <= _NF - 2.0), val, 0.0)


def _fused_kernel(*refs):
    blk_refs = refs[:_NIN]
    out_ref = refs[_NIN]
    row_sc, col_sc, md_sc = refs[_NIN + 1:]
    s = pl.program_id(0)

    @pl.when(s < _GD)
    def _gather():
        r_io = jax.lax.broadcasted_iota(jnp.int32, (_BD, _BD), 0)
        c_io = jax.lax.broadcasted_iota(jnp.int32, (_BD, _BD), 1)
        eye = r_io == c_io
        for k, bref in enumerate(blk_refs):
            dblk = jnp.where(eye, bref[...], 0.0)
            base = (s * _NIN + k) * _BD
            row_sc[:, pl.ds(base, _BD)] = jnp.sum(dblk, axis=0, keepdims=True)
            col_sc[pl.ds(base, _BD), :] = jnp.sum(dblk, axis=1, keepdims=True)

    @pl.when(s == _RANK_STEP)
    def _threshold():
        d_row = row_sc[...]
        ones = jnp.ones((_N, 1), dtype=jnp.bfloat16)
        thr = jnp.float32(0.0)
        for ci in range(_N // _C):
            d_col = col_sc[ci * _C:(ci + 1) * _C, :]
            lt = (d_row < d_col).astype(jnp.bfloat16)
            rank = jax.lax.dot_general(
                lt, ones, (((1,), (0,)), ((), ())),
                preferred_element_type=jnp.float32)
            cval = 0.5 * (_w_of_k(rank - 1.0) + _w_of_k(rank))
            thr = thr + jnp.sum(d_col * cval)
        md_sc[...] = jnp.where(d_row > thr, 0.0, d_row)

    @pl.when(s > _RANK_STEP)
    def _write():
        slab = s - _RANK_STEP - 1
        r_io = jax.lax.broadcasted_iota(jnp.int32, (_BR, _N), 0)
        c_io = jax.lax.broadcasted_iota(jnp.int32, (_BR, _N), 1)
        mask = c_io == r_io + slab * _BR
        out_ref[...] = jnp.where(mask, md_sc[...], 0.0)


def _make_in_map(k):
    last = (_GD - 1) * _NIN + k

    def in_map(s):
        i = jnp.minimum(s * _NIN + k, last)
        return (i, i)

    return in_map


def _out_map(s):
    return (jnp.maximum(s - _RANK_STEP - 1, 0), 0)


@jax.jit
def kernel(diagonal_matrix):
    return pl.pallas_call(
        _fused_kernel,
        grid=(_STEPS,),
        in_specs=[
            pl.BlockSpec((_BD, _BD), _make_in_map(k)) for k in range(_NIN)
        ],
        out_specs=pl.BlockSpec((_BR, _N), _out_map),
        out_shape=jax.ShapeDtypeStruct((_N, _N), jnp.float32),
        scratch_shapes=[
            pltpu.VMEM((1, _N), jnp.float32),
            pltpu.VMEM((_N, 1), jnp.float32),
            pltpu.VMEM((1, _N), jnp.float32),
        ],
        compiler_params=pltpu.CompilerParams(
            dimension_semantics=("arbitrary",),
        ),
    )(*([diagonal_matrix] * _NIN))


# speedup vs baseline: 3.4247x; 1.0316x over previous
import math

import jax
import jax.numpy as jnp
from jax.experimental import pallas as pl
from jax.experimental.pallas import tpu as pltpu

_N = 4096
_T = 0.1
_BD = 256
_NIN = 8
_GD = _N // (_BD * _NIN)
_C = 1024
_BR = 512
_GR = _N // _BR
_RANK_STEP = _GD
_STEPS = _GD + 1 + _GR

_BETA = 2.0 / (_N * _T)
_ZS = sum(math.exp(-_BETA * min(k + 1, _N - 1 - k)) for k in range(_N - 1))
_INV_ZS = 1.0 / _ZS
_NF = float(_N)


def _w_of_k(k):
    kk = jnp.minimum(k + 1.0, _NF - 1.0 - k)
    val = jnp.exp(-_BETA * kk) * _INV_ZS
    return jnp.where((k >= 0.0) & (k <= _NF - 2.0), val, 0.0)


def _fused_kernel(*refs):
    blk_refs = refs[:_NIN]
    out_ref = refs[_NIN]
    row_sc, col_sc, md_sc = refs[_NIN + 1:]
    s = pl.program_id(0)

    @pl.when(s < _GD)
    def _gather():
        r_io = jax.lax.broadcasted_iota(jnp.int32, (_BD, _BD), 0)
        c_io = jax.lax.broadcasted_iota(jnp.int32, (_BD, _BD), 1)
        eye = r_io == c_io
        for k, bref in enumerate(blk_refs):
            dblk = jnp.where(eye, bref[...], 0.0)
            base = (s * _NIN + k) * _BD
            row_sc[:, pl.ds(base, _BD)] = jnp.sum(dblk, axis=0, keepdims=True)
            col_sc[pl.ds(base, _BD), :] = jnp.sum(dblk, axis=1, keepdims=True)

    @pl.when(s == _RANK_STEP)
    def _threshold():
        d_row = row_sc[...]
        ones = jnp.ones((_N, 1), dtype=jnp.bfloat16)
        thr = jnp.float32(0.0)
        for ci in range(_N // _C):
            d_col = col_sc[ci * _C:(ci + 1) * _C, :]
            lt = (d_row < d_col).astype(jnp.float32)
            rank = jnp.sum(lt, axis=1, keepdims=True)
            cval = 0.5 * (_w_of_k(rank - 1.0) + _w_of_k(rank))
            thr = thr + jnp.sum(d_col * cval)
        md_sc[...] = jnp.where(d_row > thr, 0.0, d_row)

    @pl.when(s > _RANK_STEP)
    def _write():
        slab = s - _RANK_STEP - 1
        r_io = jax.lax.broadcasted_iota(jnp.int32, (_BR, _N), 0)
        c_io = jax.lax.broadcasted_iota(jnp.int32, (_BR, _N), 1)
        mask = c_io == r_io + slab * _BR
        out_ref[...] = jnp.where(mask, md_sc[...], 0.0)


def _make_in_map(k):
    last = (_GD - 1) * _NIN + k

    def in_map(s):
        i = jnp.minimum(s * _NIN + k, last)
        return (i, i)

    return in_map


def _out_map(s):
    return (jnp.maximum(s - _RANK_STEP - 1, 0), 0)


@jax.jit
def kernel(diagonal_matrix):
    return pl.pallas_call(
        _fused_kernel,
        grid=(_STEPS,),
        in_specs=[
            pl.BlockSpec((_BD, _BD), _make_in_map(k)) for k in range(_NIN)
        ],
        out_specs=pl.BlockSpec((_BR, _N), _out_map),
        out_shape=jax.ShapeDtypeStruct((_N, _N), jnp.float32),
        scratch_shapes=[
            pltpu.VMEM((1, _N), jnp.float32),
            pltpu.VMEM((_N, 1), jnp.float32),
            pltpu.VMEM((1, _N), jnp.float32),
        ],
        compiler_params=pltpu.CompilerParams(
            dimension_semantics=("arbitrary",),
        ),
    )(*([diagonal_matrix] * _NIN))
